# Initial kernel scaffold; baseline (speedup 1.0000x reference)
import functools

import jax
import jax.numpy as jnp
from jax import lax
from jax.experimental import pallas as pl
from jax.experimental.pallas import tpu as pltpu
from jax.experimental.pallas import tpu_sc as plsc

_BATCH = 16384
_HIST = 50
_NDIM = 32

_NC = 2
_NS = 16
_NW = _NC * _NS

_TRIALS_PER_W = _BATCH // _NW
_C = 16
_NCHUNK = _TRIALS_PER_W // _C
_IDX_PER_CHUNK = _C * _HIST
_G = 80
_NG = _IDX_PER_CHUNK // _G
_OUT_PER_CHUNK = _C * _NDIM * _HIST


def _build_sc_kernel():
    mesh = plsc.VectorSubcoreMesh(core_axis_name="c", subcore_axis_name="s")

    @functools.partial(
        pl.kernel,
        mesh=mesh,
        out_type=jax.ShapeDtypeStruct((_BATCH * _NDIM * _HIST,), jnp.float32),
        scratch_types=[
            pltpu.VMEM((_NG, _G), jnp.int32),
            pltpu.VMEM((_NG, _G), jnp.int32),
            pltpu.VMEM((_IDX_PER_CHUNK, _NDIM), jnp.float32),
            pltpu.VMEM((_IDX_PER_CHUNK, _NDIM), jnp.float32),
            pltpu.VMEM((_OUT_PER_CHUNK,), jnp.float32),
            pltpu.VMEM((_OUT_PER_CHUNK,), jnp.float32),
            pltpu.SemaphoreType.DMA,
            pltpu.SemaphoreType.DMA,
            pltpu.SemaphoreType.DMA,
            pltpu.SemaphoreType.DMA,
        ],
    )
    def sc_kernel(idx_hbm, z_hbm, out_hbm, idx0, idx1, rows0, rows1,
                  ob0, ob1, sg0, sg1, so0, so1):
        wid = lax.axis_index("s") * _NC + lax.axis_index("c")
        idx_row_base = wid * (_TRIALS_PER_W * _HIST // _G)
        out_base = wid * (_TRIALS_PER_W * _NDIM * _HIST)

        iota_h = jnp.arange(16, dtype=jnp.int32) * _HIST

        def start(g, idx_v, rows_v, sg):
            row0 = idx_row_base + g * _NG
            pltpu.sync_copy(idx_hbm.at[pl.ds(row0, _NG)], idx_v)
            for j in range(_NG):
                pltpu.async_copy(
                    z_hbm.at[idx_v.at[j]],
                    rows_v.at[pl.ds(j * _G, _G)],
                    sg,
                )

        def wait_gather(rows_v, sg):
            pltpu.make_async_copy(
                z_hbm.at[pl.ds(0, _IDX_PER_CHUNK)], rows_v, sg
            ).wait()

        def transpose(rows_v, out_v):
            def trial(b, carry):
                ob = b * (_NDIM * _HIST)
                rb = b * _HIST
                for h in range(_HIST):
                    r = rb + h
                    lo = rows_v[r, pl.ds(0, 16)]
                    hi = rows_v[r, pl.ds(16, 16)]
                    base = ob + h
                    plsc.store_scatter(out_v, [iota_h + base], lo)
                    plsc.store_scatter(out_v, [iota_h + (base + 16 * _HIST)], hi)
                return carry

            lax.fori_loop(0, _C, trial, 0)

        def start_out(g, out_v, so):
            pltpu.async_copy(
                out_v,
                out_hbm.at[pl.ds(out_base + g * _OUT_PER_CHUNK, _OUT_PER_CHUNK)],
                so,
            )

        def wait_out(out_v, so):
            pltpu.make_async_copy(
                out_hbm.at[pl.ds(0, _OUT_PER_CHUNK)], out_v, so
            ).wait()

        start(0, idx0, rows0, sg0)

        def pair(p, carry):
            g0 = 2 * p
            g1 = g0 + 1
            start(g1, idx1, rows1, sg1)
            wait_gather(rows0, sg0)

            @pl.when(p > 0)
            def _():
                wait_out(ob0, so0)

            transpose(rows0, ob0)
            start_out(g0, ob0, so0)

            @pl.when(p < _NCHUNK // 2 - 1)
            def _():
                start(g0 + 2, idx0, rows0, sg0)

            wait_gather(rows1, sg1)

            @pl.when(p > 0)
            def _():
                wait_out(ob1, so1)

            transpose(rows1, ob1)
            start_out(g1, ob1, so1)
            return carry

        lax.fori_loop(0, _NCHUNK // 2, pair, 0)
        wait_out(ob0, so0)
        wait_out(ob1, so1)

    return sc_kernel


_SC_KERNEL = _build_sc_kernel()


@jax.jit
def kernel(inputs, z):
    idx = jnp.reshape(inputs, (_BATCH * _HIST // _G, _G))
    out_flat = _SC_KERNEL(idx, z)
    return jnp.reshape(out_flat, (_BATCH, _NDIM, _HIST))


# speedup vs baseline: 1.6997x; 1.6997x over previous
import functools

import jax
import jax.numpy as jnp
from jax import lax
from jax.experimental import pallas as pl
from jax.experimental.pallas import tpu as pltpu
from jax.experimental.pallas import tpu_sc as plsc

_BATCH = 16384
_HIST = 50
_NDIM = 32

_NC = 2
_NS = 16
_NW = _NC * _NS

_TRIALS_PER_W = _BATCH // _NW
_C = 16
_NCHUNK = _TRIALS_PER_W // _C
_IDX_PER_CHUNK = _C * _HIST
_G = 80
_NG = _IDX_PER_CHUNK // _G
_OUT_PER_CHUNK = _C * _NDIM * _HIST


def _build_sc_kernel():
    mesh = plsc.VectorSubcoreMesh(core_axis_name="c", subcore_axis_name="s")

    @functools.partial(
        pl.kernel,
        mesh=mesh,
        out_type=jax.ShapeDtypeStruct((_BATCH * _NDIM * _HIST,), jnp.float32),
        compiler_params=pltpu.CompilerParams(
            needs_layout_passes=False, use_tc_tiling_on_sc=False
        ),
        scratch_types=[
            pltpu.VMEM((_IDX_PER_CHUNK,), jnp.int32),
            pltpu.VMEM((_IDX_PER_CHUNK,), jnp.int32),
            pltpu.VMEM((_IDX_PER_CHUNK, _NDIM), jnp.float32),
            pltpu.VMEM((_IDX_PER_CHUNK, _NDIM), jnp.float32),
            pltpu.VMEM((_OUT_PER_CHUNK,), jnp.float32),
            pltpu.VMEM((_OUT_PER_CHUNK,), jnp.float32),
            pltpu.SemaphoreType.DMA,
            pltpu.SemaphoreType.DMA,
            pltpu.SemaphoreType.DMA,
            pltpu.SemaphoreType.DMA,
        ],
    )
    def sc_kernel(idx_hbm, z_hbm, out_hbm, idx0, idx1, rows0, rows1,
                  ob0, ob1, sg0, sg1, so0, so1):
        wid = lax.axis_index("s") * _NC + lax.axis_index("c")
        idx_base = wid * (_TRIALS_PER_W * _HIST)
        out_base = wid * (_TRIALS_PER_W * _NDIM * _HIST)

        iota_h = jnp.arange(16, dtype=jnp.int32) * _HIST

        def start(g, idx_v, rows_v, sg):
            i0 = idx_base + g * _IDX_PER_CHUNK
            pltpu.sync_copy(idx_hbm.at[pl.ds(i0, _IDX_PER_CHUNK)], idx_v)
            for j in range(_NG):
                pltpu.async_copy(
                    z_hbm.at[idx_v.at[pl.ds(j * _G, _G)]],
                    rows_v.at[pl.ds(j * _G, _G)],
                    sg,
                )

        def wait_gather(rows_v, sg):
            pltpu.make_async_copy(
                z_hbm.at[pl.ds(0, _IDX_PER_CHUNK)], rows_v, sg
            ).wait()

        def transpose(rows_v, out_v):
            def trial(b, carry):
                ob = b * (_NDIM * _HIST)
                rb = b * _HIST
                for h in range(_HIST):
                    r = rb + h
                    lo = rows_v[r, pl.ds(0, 16)]
                    hi = rows_v[r, pl.ds(16, 16)]
                    base = ob + h
                    plsc.store_scatter(out_v, [iota_h + base], lo)
                    plsc.store_scatter(out_v, [iota_h + (base + 16 * _HIST)], hi)
                return carry

            lax.fori_loop(0, _C, trial, 0)

        def start_out(g, out_v, so):
            pltpu.async_copy(
                out_v,
                out_hbm.at[pl.ds(out_base + g * _OUT_PER_CHUNK, _OUT_PER_CHUNK)],
                so,
            )

        def wait_out(out_v, so):
            pltpu.make_async_copy(
                out_hbm.at[pl.ds(0, _OUT_PER_CHUNK)], out_v, so
            ).wait()

        start(0, idx0, rows0, sg0)

        def pair(p, carry):
            g0 = 2 * p
            g1 = g0 + 1
            start(g1, idx1, rows1, sg1)
            wait_gather(rows0, sg0)

            @pl.when(p > 0)
            def _():
                wait_out(ob0, so0)

            transpose(rows0, ob0)
            start_out(g0, ob0, so0)

            @pl.when(p < _NCHUNK // 2 - 1)
            def _():
                start(g0 + 2, idx0, rows0, sg0)

            wait_gather(rows1, sg1)

            @pl.when(p > 0)
            def _():
                wait_out(ob1, so1)

            transpose(rows1, ob1)
            start_out(g1, ob1, so1)
            return carry

        lax.fori_loop(0, _NCHUNK // 2, pair, 0)
        wait_out(ob0, so0)
        wait_out(ob1, so1)

    return sc_kernel


_SC_KERNEL = _build_sc_kernel()


@jax.jit
def kernel(inputs, z):
    idx = jnp.reshape(inputs, (_BATCH * _HIST,))
    out_flat = _SC_KERNEL(idx, z)
    return jnp.reshape(out_flat, (_BATCH, _NDIM, _HIST))
